# baseline (speedup 1.0000x reference)
import jax
import jax.numpy as jnp
from jax import lax
from jax.experimental import pallas as pl
from jax.experimental.pallas import tpu as pltpu

N_NODES = 10000
N_EDGES = 160000
D_IN = 256
D_HID = 1024
D_OUT = 256
BN_EPS = 1e-5

R = 400
NBLK = N_NODES // R


def _phase1_body(eps_ref, x_ref, aggA_ref, aggB_ref, W1_ref, b1_ref,
                 h1_ref, s1_ref, s2_ref):
    i = pl.program_id(0)
    scale = 1.0 + eps_ref[0, 0]
    hinA = scale * x_ref[:, :128] + aggA_ref[...]
    hinB = scale * x_ref[:, 128:] + aggB_ref[...]
    h1 = (jnp.dot(hinA, W1_ref[:128, :], preferred_element_type=jnp.float32)
          + jnp.dot(hinB, W1_ref[128:, :], preferred_element_type=jnp.float32)
          + b1_ref[...])
    h1_ref[...] = h1
    ps1 = jnp.sum(h1, axis=0, keepdims=True)
    ps2 = jnp.sum(h1 * h1, axis=0, keepdims=True)

    @pl.when(i == 0)
    def _():
        s1_ref[...] = ps1
        s2_ref[...] = ps2

    @pl.when(i != 0)
    def _():
        s1_ref[...] += ps1
        s2_ref[...] += ps2


def _phase2_body(h1_ref, s1_ref, s2_ref, g1_ref, beta1_ref, W2_ref, b2_ref,
                 h2_ref, t1_ref, t2_ref):
    i = pl.program_id(0)
    n = jnp.float32(N_NODES)
    mu = s1_ref[...] / n
    var = s2_ref[...] / n - mu * mu
    rstd = lax.rsqrt(var + BN_EPS)
    a = (h1_ref[...] - mu) * (rstd * g1_ref[...]) + beta1_ref[...]
    a = jnp.maximum(a, 0.0)
    h2 = jnp.dot(a, W2_ref[...], preferred_element_type=jnp.float32) + b2_ref[...]
    h2_ref[...] = h2
    ps1 = jnp.sum(h2, axis=0, keepdims=True)
    ps2 = jnp.sum(h2 * h2, axis=0, keepdims=True)

    @pl.when(i == 0)
    def _():
        t1_ref[...] = ps1
        t2_ref[...] = ps2

    @pl.when(i != 0)
    def _():
        t1_ref[...] += ps1
        t2_ref[...] += ps2


def _phase3_body(h2_ref, t1_ref, t2_ref, g2_ref, beta2_ref, out_ref):
    n = jnp.float32(N_NODES)
    mu = t1_ref[...] / n
    var = t2_ref[...] / n - mu * mu
    rstd = lax.rsqrt(var + BN_EPS)
    out_ref[...] = (h2_ref[...] - mu) * (rstd * g2_ref[...]) + beta2_ref[...]


def _aggregate(x, src, dst):
    msgs = jnp.take(x, src, axis=0)
    agg = jax.ops.segment_sum(msgs, dst, num_segments=N_NODES)
    return agg[:, :128], agg[:, 128:]


def kernel(x, edge_index, eps, W1, b1, g1, beta1, W2, b2, g2, beta2):
    src = edge_index[0].astype(jnp.int32)
    dst = edge_index[1].astype(jnp.int32)
    aggA, aggB = _aggregate(x, src, dst)

    eps2 = eps.reshape(1, 1)
    b1r = b1.reshape(1, D_HID)
    g1r = g1.reshape(1, D_HID)
    beta1r = beta1.reshape(1, D_HID)
    b2r = b2.reshape(1, D_OUT)
    g2r = g2.reshape(1, D_OUT)
    beta2r = beta2.reshape(1, D_OUT)

    full = lambda shape: pl.BlockSpec(shape, lambda i: (0,) * len(shape))
    rowblk = lambda c: pl.BlockSpec((R, c), lambda i: (i, 0))

    h1, s1, s2 = pl.pallas_call(
        _phase1_body,
        grid=(NBLK,),
        in_specs=[full((1, 1)), rowblk(D_IN), rowblk(128), rowblk(128),
                  full((D_IN, D_HID)), full((1, D_HID))],
        out_specs=[rowblk(D_HID), full((1, D_HID)), full((1, D_HID))],
        out_shape=[jax.ShapeDtypeStruct((N_NODES, D_HID), jnp.float32),
                   jax.ShapeDtypeStruct((1, D_HID), jnp.float32),
                   jax.ShapeDtypeStruct((1, D_HID), jnp.float32)],
    )(eps2, x, aggA, aggB, W1, b1r)

    h2, t1, t2 = pl.pallas_call(
        _phase2_body,
        grid=(NBLK,),
        in_specs=[rowblk(D_HID), full((1, D_HID)), full((1, D_HID)),
                  full((1, D_HID)), full((1, D_HID)),
                  full((D_HID, D_OUT)), full((1, D_OUT))],
        out_specs=[rowblk(D_OUT), full((1, D_OUT)), full((1, D_OUT))],
        out_shape=[jax.ShapeDtypeStruct((N_NODES, D_OUT), jnp.float32),
                   jax.ShapeDtypeStruct((1, D_OUT), jnp.float32),
                   jax.ShapeDtypeStruct((1, D_OUT), jnp.float32)],
    )(h1, s1, s2, g1r, beta1r, W2, b2r)

    out = pl.pallas_call(
        _phase3_body,
        grid=(NBLK,),
        in_specs=[rowblk(D_OUT), full((1, D_OUT)), full((1, D_OUT)),
                  full((1, D_OUT)), full((1, D_OUT))],
        out_specs=rowblk(D_OUT),
        out_shape=jax.ShapeDtypeStruct((N_NODES, D_OUT), jnp.float32),
    )(h2, t1, t2, g2r, beta2r)
    return out


# speedup vs baseline: 3.1730x; 3.1730x over previous
import functools

import jax
import jax.numpy as jnp
from jax import lax
from jax.experimental import pallas as pl
from jax.experimental.pallas import tpu as pltpu
from jax.experimental.pallas import tpu_sc as plsc

N_NODES = 10000
N_EDGES = 160000
D_IN = 256
D_HID = 1024
D_OUT = 256
BN_EPS = 1e-5

R = 400
NBLK = N_NODES // R


def _phase1_body(eps_ref, x_ref, aggA_ref, aggB_ref, W1_ref, b1_ref,
                 h1_ref, s1_ref, s2_ref):
    i = pl.program_id(0)
    scale = 1.0 + eps_ref[0, 0]
    hinA = scale * x_ref[:, :128] + aggA_ref[...]
    hinB = scale * x_ref[:, 128:] + aggB_ref[...]
    h1 = (jnp.dot(hinA, W1_ref[:128, :], preferred_element_type=jnp.float32)
          + jnp.dot(hinB, W1_ref[128:, :], preferred_element_type=jnp.float32)
          + b1_ref[...])
    h1_ref[...] = h1
    ps1 = jnp.sum(h1, axis=0, keepdims=True)
    ps2 = jnp.sum(h1 * h1, axis=0, keepdims=True)

    @pl.when(i == 0)
    def _():
        s1_ref[...] = ps1
        s2_ref[...] = ps2

    @pl.when(i != 0)
    def _():
        s1_ref[...] += ps1
        s2_ref[...] += ps2


def _phase2_body(h1_ref, s1_ref, s2_ref, g1_ref, beta1_ref, W2_ref, b2_ref,
                 h2_ref, t1_ref, t2_ref):
    i = pl.program_id(0)
    n = jnp.float32(N_NODES)
    mu = s1_ref[...] / n
    var = s2_ref[...] / n - mu * mu
    rstd = lax.rsqrt(var + BN_EPS)
    a = (h1_ref[...] - mu) * (rstd * g1_ref[...]) + beta1_ref[...]
    a = jnp.maximum(a, 0.0)
    h2 = jnp.dot(a, W2_ref[...], preferred_element_type=jnp.float32) + b2_ref[...]
    h2_ref[...] = h2
    ps1 = jnp.sum(h2, axis=0, keepdims=True)
    ps2 = jnp.sum(h2 * h2, axis=0, keepdims=True)

    @pl.when(i == 0)
    def _():
        t1_ref[...] = ps1
        t2_ref[...] = ps2

    @pl.when(i != 0)
    def _():
        t1_ref[...] += ps1
        t2_ref[...] += ps2


def _phase3_body(h2_ref, t1_ref, t2_ref, g2_ref, beta2_ref, out_ref):
    n = jnp.float32(N_NODES)
    mu = t1_ref[...] / n
    var = t2_ref[...] / n - mu * mu
    rstd = lax.rsqrt(var + BN_EPS)
    out_ref[...] = (h2_ref[...] - mu) * (rstd * g2_ref[...]) + beta2_ref[...]


K = 128
E_TILE = 10240
CH = E_TILE // K
N_ACC = 10240
ZROWS = N_ACC // 16
OROWS = 624


def _agg_sc_body(x2_hbm, src2_hbm, dst_hbm, zeros_hbm, out_hbm,
                 src_vm, dst_vm, rows_vm, acc_sh, sem):
    c = lax.axis_index("c")
    s = lax.axis_index("s")
    pltpu.sync_copy(src2_hbm.at[c, s], src_vm)
    pltpu.sync_copy(dst_hbm.at[s], dst_vm)
    pltpu.sync_copy(zeros_hbm, acc_sh.at[pl.ds(s * ZROWS, ZROWS)])
    plsc.subcore_barrier()

    def body(j, carry):
        pltpu.async_copy(x2_hbm.at[src_vm.at[j]], rows_vm, sem).wait()
        pltpu.sync_copy(rows_vm, acc_sh.at[dst_vm.at[j]], add=True)
        return carry

    lax.fori_loop(0, CH, body, 0)
    plsc.subcore_barrier()
    pltpu.sync_copy(acc_sh.at[pl.ds(s * OROWS, OROWS)],
                    out_hbm.at[c, pl.ds(s * OROWS, OROWS)])

    @pl.when(s == 15)
    def _():
        pltpu.sync_copy(acc_sh.at[pl.ds(16 * OROWS, N_NODES - 16 * OROWS)],
                        out_hbm.at[c, pl.ds(16 * OROWS, N_NODES - 16 * OROWS)])


@functools.partial(
    pl.kernel,
    out_type=jax.ShapeDtypeStruct((2, N_NODES, 128), jnp.float32),
    mesh=plsc.VectorSubcoreMesh(core_axis_name="c", subcore_axis_name="s"),
    scratch_types=[
        pltpu.VMEM((CH, K), jnp.int32),
        pltpu.VMEM((CH, K), jnp.int32),
        pltpu.VMEM((K, 128), jnp.float32),
        pltpu.VMEM_SHARED((N_ACC, 128), jnp.float32),
        pltpu.SemaphoreType.DMA,
    ],
)
def _agg_sc(x2_hbm, src2_hbm, dst_hbm, zeros_hbm, out_hbm,
            src_vm, dst_vm, rows_vm, acc_sh, sem):
    _agg_sc_body(x2_hbm, src2_hbm, dst_hbm, zeros_hbm, out_hbm,
                 src_vm, dst_vm, rows_vm, acc_sh, sem)


def _aggregate(x, src, dst):
    npad = E_TILE - N_EDGES // 16
    x2 = x.reshape(2 * N_NODES, 128)
    src_t = jnp.concatenate(
        [src.reshape(16, N_EDGES // 16),
         jnp.zeros((16, npad), jnp.int32)], axis=1)
    src2 = jnp.stack([src_t * 2, src_t * 2 + 1]).reshape(2, 16, CH, K)
    dst_t = jnp.concatenate(
        [dst.reshape(16, N_EDGES // 16),
         jnp.full((16, npad), N_NODES, jnp.int32)], axis=1).reshape(16, CH, K)
    zeros = jnp.zeros((ZROWS, 128), jnp.float32)
    agg = _agg_sc(x2, src2, dst_t, zeros)
    return agg[0], agg[1]


def kernel(x, edge_index, eps, W1, b1, g1, beta1, W2, b2, g2, beta2):
    src = edge_index[0].astype(jnp.int32)
    dst = edge_index[1].astype(jnp.int32)
    aggA, aggB = _aggregate(x, src, dst)

    eps2 = eps.reshape(1, 1)
    b1r = b1.reshape(1, D_HID)
    g1r = g1.reshape(1, D_HID)
    beta1r = beta1.reshape(1, D_HID)
    b2r = b2.reshape(1, D_OUT)
    g2r = g2.reshape(1, D_OUT)
    beta2r = beta2.reshape(1, D_OUT)

    full = lambda shape: pl.BlockSpec(shape, lambda i: (0,) * len(shape))
    rowblk = lambda c: pl.BlockSpec((R, c), lambda i: (i, 0))

    h1, s1, s2 = pl.pallas_call(
        _phase1_body,
        grid=(NBLK,),
        in_specs=[full((1, 1)), rowblk(D_IN), rowblk(128), rowblk(128),
                  full((D_IN, D_HID)), full((1, D_HID))],
        out_specs=[rowblk(D_HID), full((1, D_HID)), full((1, D_HID))],
        out_shape=[jax.ShapeDtypeStruct((N_NODES, D_HID), jnp.float32),
                   jax.ShapeDtypeStruct((1, D_HID), jnp.float32),
                   jax.ShapeDtypeStruct((1, D_HID), jnp.float32)],
    )(eps2, x, aggA, aggB, W1, b1r)

    h2, t1, t2 = pl.pallas_call(
        _phase2_body,
        grid=(NBLK,),
        in_specs=[rowblk(D_HID), full((1, D_HID)), full((1, D_HID)),
                  full((1, D_HID)), full((1, D_HID)),
                  full((D_HID, D_OUT)), full((1, D_OUT))],
        out_specs=[rowblk(D_OUT), full((1, D_OUT)), full((1, D_OUT))],
        out_shape=[jax.ShapeDtypeStruct((N_NODES, D_OUT), jnp.float32),
                   jax.ShapeDtypeStruct((1, D_OUT), jnp.float32),
                   jax.ShapeDtypeStruct((1, D_OUT), jnp.float32)],
    )(h1, s1, s2, g1r, beta1r, W2, b2r)

    out = pl.pallas_call(
        _phase3_body,
        grid=(NBLK,),
        in_specs=[rowblk(D_OUT), full((1, D_OUT)), full((1, D_OUT)),
                  full((1, D_OUT)), full((1, D_OUT))],
        out_specs=rowblk(D_OUT),
        out_shape=jax.ShapeDtypeStruct((N_NODES, D_OUT), jnp.float32),
    )(h2, t1, t2, g2r, beta2r)
    return out
